# Initial kernel scaffold; baseline (speedup 1.0000x reference)
import jax
import jax.numpy as jnp
from jax.experimental import pallas as pl


def kernel(coordinates, W0, b0, W1, b1, W2, b2, W3, b3, weight1, bias1, local_cellxgene_ix, genes_oi):
    raise NotImplementedError("write your pallas kernel here")


# speedup vs baseline: 1.6038x; 1.6038x over previous
import functools

import jax
import jax.numpy as jnp
import numpy as np
from jax.experimental import pallas as pl
from jax.experimental.pallas import tpu as pltpu

N_FRAG = 320000
N_CELLS = 100
NGB = 1000
D = 100
DP = 112
NSEG = N_CELLS * NGB
SEG_CHUNK = 4096
N_CHUNKS = 26
NSEG_PAD = SEG_CHUNK * N_CHUNKS
MLP_BLK = 2048
PAD_ID = 1 << 30

_WINDOW = (-10000.0, 10000.0)
_SCALE = _WINDOW[1] - _WINDOW[0]
_SHIFT = _WINDOW[0] + _SCALE / 2.0


def _mlp_body(x_ref, w0t_ref, b0_ref, w1t_ref, b1_ref, w2ta_ref, b2a_ref, out_ref):
    x = (x_ref[...] - _SHIFT) / _SCALE
    w0t = w0t_ref[...]
    h = jnp.maximum(x[:, 0:1] * w0t[0:1, :] + x[:, 1:2] * w0t[1:2, :] + b0_ref[...], 0.0)
    h = jnp.maximum(jnp.dot(h, w1t_ref[...], preferred_element_type=jnp.float32) + b1_ref[...], 0.0)
    h = jnp.maximum(jnp.dot(h, w2ta_ref[...], preferred_element_type=jnp.float32) + b2a_ref[...], 0.0)
    out_ref[...] = h


def _mlp_call(coords_p, w0t, b0, w1t, b1, w2ta, b2a, n_pad):
    grid = n_pad // MLP_BLK
    rep = lambda i: (0, 0)
    return pl.pallas_call(
        _mlp_body,
        grid=grid,
        in_specs=[
            pl.BlockSpec((MLP_BLK, 2), lambda i: (i, 0)),
            pl.BlockSpec((2, D), rep),
            pl.BlockSpec((D,), lambda i: (0,)),
            pl.BlockSpec((D, D), rep),
            pl.BlockSpec((D,), lambda i: (0,)),
            pl.BlockSpec((D, DP), rep),
            pl.BlockSpec((DP,), lambda i: (0,)),
        ],
        out_specs=pl.BlockSpec((MLP_BLK, DP), lambda i: (i, 0)),
        out_shape=jax.ShapeDtypeStruct((n_pad, DP), jnp.float32),
    )(coords_p, w0t, b0, w1t, b1, w2ta, b2a)


CB = 8
N_CELLS_PAD = 104


def _combine_body(bins_ref, vpad_ref, biasg_ref, out_ref):
    b = bins_ref[...].reshape(CB, NGB, DP)
    s = jnp.sum(b * vpad_ref[...][None], axis=-1)
    cnt = b[:, :, 100]
    out_ref[...] = s / jnp.maximum(cnt, 1.0) + biasg_ref[...][None]


def _combine_call(bins, vpad, biasg):
    return pl.pallas_call(
        _combine_body,
        grid=N_CELLS_PAD // CB,
        in_specs=[
            pl.BlockSpec((CB * NGB, DP), lambda c: (c, 0)),
            pl.BlockSpec((NGB, DP), lambda c: (0, 0)),
            pl.BlockSpec((NGB,), lambda c: (0,)),
        ],
        out_specs=pl.BlockSpec((CB, NGB), lambda c: (c, 0)),
        out_shape=jax.ShapeDtypeStruct((N_CELLS_PAD, NGB), jnp.float32),
    )(bins, vpad, biasg)


def _vtable_body(wg_ref, w3aug_ref, vpad_ref):
    vpad_ref[...] = jnp.dot(wg_ref[...], w3aug_ref[...], preferred_element_type=jnp.float32)


def _vtable_call(wg, w3aug):
    return pl.pallas_call(
        _vtable_body,
        out_shape=jax.ShapeDtypeStruct((NGB, DP), jnp.float32),
    )(wg, w3aug)


def kernel(coordinates, W0, b0, W1, b1, W2, b2, W3, b3, weight1, bias1, local_cellxgene_ix, genes_oi):
    n_pad = ((N_FRAG + MLP_BLK - 1) // MLP_BLK) * MLP_BLK
    coords_p = jnp.pad(coordinates, ((0, n_pad - N_FRAG), (0, 0)))
    lix = local_cellxgene_ix.astype(jnp.int32)
    lix_p = jnp.pad(lix, (0, n_pad - N_FRAG), constant_values=PAD_ID)

    w2ta = jnp.zeros((D, DP), jnp.float32).at[:, :D].set(W2.T)
    b2a = jnp.zeros((DP,), jnp.float32).at[:D].set(b2).at[D].set(1.0)

    h_aug = _mlp_call(coords_p, W0.T, b0, W1.T, b1, w2ta, b2a, n_pad)

    bins = jax.ops.segment_sum(h_aug, lix_p, num_segments=NSEG_PAD,
                               indices_are_sorted=True)

    wg = weight1[genes_oi]
    biasg = bias1[genes_oi]
    w3aug = jnp.zeros((D, DP), jnp.float32).at[:, :D].set(W3).at[:, D].set(b3)
    vpad = _vtable_call(wg, w3aug)
    return _combine_call(bins, vpad, biasg)[:N_CELLS]
